# baseline (speedup 1.0000x reference)
import functools

import jax
import jax.numpy as jnp
import numpy as np
from jax.experimental import pallas as pl
from jax.experimental.pallas import tpu as pltpu

_N_BINS = 15
_ROWS = 16384
_COLS = 1000
_BLOCK_ROWS = 512


def _ece_kernel(x_ref, lab_ref, bnd_ref, out_ref, acc_ref):
    i = pl.program_id(0)
    n_steps = pl.num_programs(0)

    @pl.when(i == 0)
    def _init():
        acc_ref[...] = jnp.zeros_like(acc_ref)

    x = x_ref[...]
    lab = lab_ref[0, 0, :]

    m = jnp.max(x, axis=1, keepdims=True)
    s = jnp.sum(jnp.exp(x - m), axis=1)
    conf = 1.0 / s

    col = jax.lax.broadcasted_iota(jnp.int32, x.shape, 1)
    idx = jnp.min(jnp.where(x == m, col, _COLS), axis=1)
    acc = (idx == lab).astype(jnp.float32)

    lowers = bnd_ref[:, 0:1]
    uppers = bnd_ref[:, 1:2]

    c2 = conf[None, :]
    in_bin = ((c2 > lowers) & (c2 <= uppers)).astype(jnp.float32)

    acc_ref[0, :] += jnp.sum(in_bin, axis=1)
    acc_ref[1, :] += jnp.sum(in_bin * c2, axis=1)
    acc_ref[2, :] += jnp.sum(in_bin * acc[None, :], axis=1)

    @pl.when(i == n_steps - 1)
    def _finish():
        count = acc_ref[0, :]
        conf_sum = acc_ref[1, :]
        acc_sum = acc_ref[2, :]
        prop = count / float(_ROWS)
        denom = jnp.maximum(count, 1.0)
        gaps = jnp.where(
            count > 0.0,
            jnp.abs(conf_sum / denom - acc_sum / denom) * prop,
            0.0,
        )
        out_ref[...] = jnp.sum(gaps).reshape(1, 1)


@functools.partial(jax.jit)
def _ece(logits, labels):
    g = _ROWS // _BLOCK_ROWS
    labels3 = labels.astype(jnp.int32).reshape(g, 1, _BLOCK_ROWS)
    bb = jnp.linspace(0.0, 1.0, _N_BINS + 1)
    bounds = jnp.stack([bb[:-1], bb[1:]], axis=1)
    out = pl.pallas_call(
        _ece_kernel,
        grid=(g,),
        in_specs=[
            pl.BlockSpec((_BLOCK_ROWS, _COLS), lambda i: (i, 0)),
            pl.BlockSpec((1, 1, _BLOCK_ROWS), lambda i: (i, 0, 0)),
            pl.BlockSpec((_N_BINS, 2), lambda i: (0, 0)),
        ],
        out_specs=pl.BlockSpec((1, 1), lambda i: (0, 0)),
        out_shape=jax.ShapeDtypeStruct((1, 1), jnp.float32),
        scratch_shapes=[pltpu.VMEM((3, _N_BINS), jnp.float32)],
    )(logits, labels3, bounds)
    return out.reshape(1)


def kernel(logits, labels):
    return _ece(logits, labels)


# speedup vs baseline: 1.7795x; 1.7795x over previous
import functools

import jax
import jax.numpy as jnp
from jax.experimental import pallas as pl
from jax.experimental.pallas import tpu as pltpu

_N_BINS = 15
_ROWS = 16384
_COLS = 1000
_BLOCK_ROWS = 512
_G = _ROWS // _BLOCK_ROWS


def _bins_kernel(x_ref, lab_ref, bnd_ref, out_ref):
    x = x_ref[...]
    lab = lab_ref[0]

    m = jnp.max(x, axis=1, keepdims=True)
    s = jnp.sum(jnp.exp(x - m), axis=1, keepdims=True)
    conf = 1.0 / s

    col = jax.lax.broadcasted_iota(jnp.int32, x.shape, 1)
    idx = jnp.min(jnp.where(x == m, col, _COLS), axis=1, keepdims=True)
    acc = (idx == lab).astype(jnp.float32)

    lo = bnd_ref[0:1, :]
    hi = bnd_ref[1:2, :]
    in_bin = ((conf > lo) & (conf <= hi)).astype(jnp.float32)

    out_ref[0, 0:1, :] = jnp.sum(in_bin, axis=0, keepdims=True)
    out_ref[0, 1:2, :] = jnp.sum(in_bin * conf, axis=0, keepdims=True)
    out_ref[0, 2:3, :] = jnp.sum(in_bin * acc, axis=0, keepdims=True)


def _combine_kernel(p_ref, out_ref):
    p = p_ref[...]
    count = jnp.sum(p[:, 0, :], axis=0)
    conf_sum = jnp.sum(p[:, 1, :], axis=0)
    acc_sum = jnp.sum(p[:, 2, :], axis=0)
    prop = count / float(_ROWS)
    denom = jnp.maximum(count, 1.0)
    gaps = jnp.where(
        count > 0.0,
        jnp.abs(conf_sum / denom - acc_sum / denom) * prop,
        0.0,
    )
    out_ref[...] = jnp.sum(gaps).reshape(1, 1)


@jax.jit
def _ece(logits, labels):
    labels3 = labels.astype(jnp.int32).reshape(_G, _BLOCK_ROWS, 1)
    bb = jnp.linspace(0.0, 1.0, _N_BINS + 1)
    bounds = jnp.stack(
        [
            jnp.concatenate([bb[:-1], jnp.array([2.0], jnp.float32)]),
            jnp.concatenate([bb[1:], jnp.array([2.0], jnp.float32)]),
        ],
        axis=0,
    )
    partials = pl.pallas_call(
        _bins_kernel,
        grid=(_G,),
        in_specs=[
            pl.BlockSpec((_BLOCK_ROWS, _COLS), lambda i: (i, 0)),
            pl.BlockSpec((1, _BLOCK_ROWS, 1), lambda i: (i, 0, 0)),
            pl.BlockSpec((2, 16), lambda i: (0, 0)),
        ],
        out_specs=pl.BlockSpec((1, 3, 16), lambda i: (i, 0, 0)),
        out_shape=jax.ShapeDtypeStruct((_G, 3, 16), jnp.float32),
        compiler_params=pltpu.CompilerParams(
            dimension_semantics=("parallel",),
        ),
    )(logits, labels3, bounds)
    out = pl.pallas_call(
        _combine_kernel,
        out_shape=jax.ShapeDtypeStruct((1, 1), jnp.float32),
    )(partials)
    return out.reshape(1)


def kernel(logits, labels):
    return _ece(logits, labels)


# speedup vs baseline: 1.9855x; 1.1158x over previous
import functools

import jax
import jax.numpy as jnp
from jax.experimental import pallas as pl
from jax.experimental.pallas import tpu as pltpu

_N_BINS = 15
_ROWS = 16384
_COLS = 1000
_BLOCK_ROWS = 2048
_G = _ROWS // _BLOCK_ROWS


def _bins_kernel(x_ref, lab_ref, bnd_ref, out_ref):
    x = x_ref[...]
    lab = lab_ref[0]

    m = jnp.max(x, axis=1, keepdims=True)
    s = jnp.sum(jnp.exp(x - m), axis=1, keepdims=True)
    conf = 1.0 / s

    col = jax.lax.broadcasted_iota(jnp.int32, x.shape, 1)
    idx = jnp.min(jnp.where(x == m, col, _COLS), axis=1, keepdims=True)
    acc = (idx == lab).astype(jnp.float32)

    lo = bnd_ref[0:1, :]
    hi = bnd_ref[1:2, :]
    in_bin = ((conf > lo) & (conf <= hi)).astype(jnp.float32)

    out_ref[0, 0:1, :] = jnp.sum(in_bin, axis=0, keepdims=True)
    out_ref[0, 1:2, :] = jnp.sum(in_bin * conf, axis=0, keepdims=True)
    out_ref[0, 2:3, :] = jnp.sum(in_bin * acc, axis=0, keepdims=True)


def _combine_kernel(p_ref, out_ref):
    p = p_ref[...]
    count = jnp.sum(p[:, 0, :], axis=0)
    conf_sum = jnp.sum(p[:, 1, :], axis=0)
    acc_sum = jnp.sum(p[:, 2, :], axis=0)
    prop = count / float(_ROWS)
    denom = jnp.maximum(count, 1.0)
    gaps = jnp.where(
        count > 0.0,
        jnp.abs(conf_sum / denom - acc_sum / denom) * prop,
        0.0,
    )
    out_ref[...] = jnp.sum(gaps).reshape(1, 1)


@jax.jit
def _ece(logits, labels):
    labels3 = labels.astype(jnp.int32).reshape(_G, _BLOCK_ROWS, 1)
    bb = jnp.linspace(0.0, 1.0, _N_BINS + 1)
    bounds = jnp.stack(
        [
            jnp.concatenate([bb[:-1], jnp.array([2.0], jnp.float32)]),
            jnp.concatenate([bb[1:], jnp.array([2.0], jnp.float32)]),
        ],
        axis=0,
    )
    partials = pl.pallas_call(
        _bins_kernel,
        grid=(_G,),
        in_specs=[
            pl.BlockSpec((_BLOCK_ROWS, _COLS), lambda i: (i, 0)),
            pl.BlockSpec((1, _BLOCK_ROWS, 1), lambda i: (i, 0, 0)),
            pl.BlockSpec((2, 16), lambda i: (0, 0)),
        ],
        out_specs=pl.BlockSpec((1, 3, 16), lambda i: (i, 0, 0)),
        out_shape=jax.ShapeDtypeStruct((_G, 3, 16), jnp.float32),
        compiler_params=pltpu.CompilerParams(
            dimension_semantics=("parallel",),
        ),
    )(logits, labels3, bounds)
    out = pl.pallas_call(
        _combine_kernel,
        out_shape=jax.ShapeDtypeStruct((1, 1), jnp.float32),
    )(partials)
    return out.reshape(1)


def kernel(logits, labels):
    return _ece(logits, labels)
